# baseline (speedup 1.0000x reference)
import jax
import jax.numpy as jnp
from jax import lax
from jax.experimental import pallas as pl
from jax.experimental.pallas import tpu as pltpu

_VMEM_LIMIT_BYTES = 56 * 1024 * 1024
_TM = 512
_TW = 1024
_DEPTH = 3
_NBUF = 6


def _round_up(v, m):
    return ((v + m - 1) // m) * m


def _schedule(nm):
    per_group = []
    for g in range(nm):
        blocks = []
        if g % 2 == 1:
            J = (g - 1) // 2
            for i in range(g):
                blocks.append((i, J))
            for J2 in range((g - 1) // 2):
                blocks.append((g, J2))
            blocks.append((g, J))
        else:
            for J2 in range(g // 2):
                blocks.append((g, J2))
        per_group.append(blocks)
    return per_group


def _make_kernel(n_pad, nhid, nm):
    sched = _schedule(nm)

    def kern(x_ref, w_ref, b_ref, adj_hbm, out_hbm, abuf, wbuf, h_scr,
             rsem, wsem):
        def start_read(s):
            pltpu.make_async_copy(
                adj_hbm.at[pl.ds(s * _TM, _TM), :],
                abuf.at[s % _DEPTH], rsem.at[s % _DEPTH]).start()

        def wait_read(s):
            pltpu.make_async_copy(
                abuf.at[s % _DEPTH], abuf.at[s % _DEPTH],
                rsem.at[s % _DEPTH]).wait()

        def wait_write(slot):
            pltpu.make_async_copy(
                wbuf.at[slot], wbuf.at[slot], wsem.at[slot]).wait()

        for d in range(min(_DEPTH, nm)):
            start_read(d)

        wcnt = 0
        for g in range(nm):
            wait_read(g)
            a = abuf.at[g % _DEPTH]
            acc = jnp.dot(a[...], x_ref[...],
                          preferred_element_type=jnp.float32)
            z = jnp.dot(acc, w_ref[...],
                        preferred_element_type=jnp.float32) + b_ref[...]
            h_scr[pl.ds(g * _TM, _TM), :] = jnp.maximum(z, jnp.float32(0.0))
            if g + _DEPTH < nm:
                start_read(g + _DEPTH)

            for (i, J) in sched[g]:
                slot = wcnt % _NBUF
                if wcnt >= _NBUF:
                    wait_write(slot)
                hi = h_scr[pl.ds(i * _TM, _TM), :]
                hj = h_scr[pl.ds(J * _TW, _TW), :]
                wbuf[slot] = lax.dot_general(
                    hi, hj, dimension_numbers=(((1,), (1,)), ((), ())),
                    preferred_element_type=jnp.float32)
                pltpu.make_async_copy(
                    wbuf.at[slot],
                    out_hbm.at[pl.ds(i * _TM, _TM), pl.ds(J * _TW, _TW)],
                    wsem.at[slot]).start()
                wcnt += 1

        for slot in range(min(wcnt, _NBUF)):
            wait_write(slot)

    return kern


def kernel(x, adj, weight, bias):
    n, nhid = x.shape
    assert adj.shape == (n, n)
    assert weight.shape == (nhid, nhid)
    assert bias.shape == (nhid,)

    x = x.astype(jnp.float32)
    adj = adj.astype(jnp.float32)
    weight = weight.astype(jnp.float32)
    bias = bias.astype(jnp.float32)

    n_pad = _round_up(n, _TW)
    if n_pad != n:
        adj_p = jnp.zeros((n_pad, n_pad), jnp.float32).at[:n, :n].set(adj)
        x_p = jnp.zeros((n_pad, nhid), jnp.float32).at[:n, :].set(x)
    else:
        adj_p, x_p = adj, x

    nm = n_pad // _TM
    bias2d = bias.reshape(1, nhid)

    out_p = pl.pallas_call(
        _make_kernel(n_pad, nhid, nm),
        out_shape=jax.ShapeDtypeStruct((n_pad, n_pad), jnp.float32),
        grid=(),
        in_specs=[
            pl.BlockSpec(memory_space=pltpu.MemorySpace.VMEM),
            pl.BlockSpec(memory_space=pltpu.MemorySpace.VMEM),
            pl.BlockSpec(memory_space=pltpu.MemorySpace.VMEM),
            pl.BlockSpec(memory_space=pl.ANY),
        ],
        out_specs=pl.BlockSpec(memory_space=pl.ANY),
        scratch_shapes=[
            pltpu.VMEM((_DEPTH, _TM, n_pad), jnp.float32),
            pltpu.VMEM((_NBUF, _TM, _TW), jnp.float32),
            pltpu.VMEM((n_pad, nhid), jnp.float32),
            pltpu.SemaphoreType.DMA((_DEPTH,)),
            pltpu.SemaphoreType.DMA((_NBUF,)),
        ],
        compiler_params=pltpu.CompilerParams(
            vmem_limit_bytes=_VMEM_LIMIT_BYTES,
        ),
        cost_estimate=pl.CostEstimate(
            flops=4 * n_pad * n_pad * nhid,
            transcendentals=0,
            bytes_accessed=4 * (2 * n_pad * n_pad + 2 * n_pad * nhid),
        ),
    )(x_p, weight, bias2d, adj_p)

    if n_pad != n:
        return out_p[:n, :n]
    return out_p
